# Initial kernel scaffold; baseline (speedup 1.0000x reference)
import jax
import jax.numpy as jnp
from jax.experimental import pallas as pl


def kernel(Cls, Shape, Offset):
    raise NotImplementedError("write your pallas kernel here")


# speedup vs baseline: 3.8245x; 3.8245x over previous
import numpy as np
import jax
import jax.numpy as jnp
from jax.experimental import pallas as pl

_TOPK = 60
_THRESHOLD = 0.15
_NMS_THRESHOLD = 0.05
_NMS_TOPK = 20
_CROP = 96.0


def _make_body(bs, d, h, w):
    N = d * h * w
    stride_z = _CROP / d
    stride_y = _CROP / h
    stride_x = _CROP / w

    def body(cls_ref, off_ref, shp_ref, anch_ref, out_ref):
        S = cls_ref[...]
        iota = jax.lax.broadcasted_iota(jnp.int32, (bs, N), 1)
        lane60 = jax.lax.broadcasted_iota(jnp.int32, (bs, _TOPK), 1)

        def topk_step(k, carry):
            s, r, ts = carry
            m = jnp.max(s, axis=1, keepdims=True)
            idx = jnp.min(jnp.where(s == m, iota, N), axis=1, keepdims=True)
            sel = iota == idx
            s = jnp.where(sel, -jnp.inf, s)
            r = jnp.where(sel, k, r)
            ts = jnp.where(lane60 == k, m, ts)
            return s, r, ts

        R0 = jnp.full((bs, N), _TOPK, jnp.int32)
        ts0 = jnp.zeros((bs, _TOPK), jnp.float32)
        _, R, ts_logit = jax.lax.fori_loop(0, _TOPK, topk_step, (S, R0, ts0))

        anch = anch_ref[...]
        iota60N = jax.lax.broadcasted_iota(jnp.int32, (_TOPK, N), 0)
        dn = (((1,), (1,)), ((), ()))
        gs = []
        for b in range(bs):
            oh = (R[b:b + 1, :] == iota60N).astype(jnp.float32)
            vals = jnp.concatenate([anch, off_ref[b], shp_ref[b]], axis=0)
            g = jax.lax.dot_general(vals, oh, dn,
                                    preferred_element_type=jnp.float32)
            gs.append(g[None])
        G = jnp.concatenate(gs, axis=0)

        az, ay, ax = G[:, 0, :], G[:, 1, :], G[:, 2, :]
        oz, oy, ox = G[:, 3, :], G[:, 4, :], G[:, 5, :]
        pz, py, px = G[:, 6, :], G[:, 7, :], G[:, 8, :]
        cz = (az + oz) * stride_z
        cy = (ay + oy) * stride_y
        cx = (ax + ox) * stride_x
        dz = 2.0 * pz
        dy = 2.0 * py
        dx = 2.0 * px

        sig = jax.nn.sigmoid(ts_logit)
        loz, hiz = cz - dz * 0.5, cz + dz * 0.5
        loy, hiy = cy - dy * 0.5, cy + dy * 0.5
        lox, hix = cx - dx * 0.5, cx + dx * 0.5
        vols = dz * dy * dx

        sup = jnp.logical_not(sig > _THRESHOLD)
        kcount = jnp.zeros((bs, 1), jnp.int32)
        neg1 = jnp.full((bs, _TOPK), -1.0, jnp.float32)
        outs = [neg1] * 8

        for i in range(_TOPK):
            sup_i = sup[:, i:i + 1]
            take = jnp.logical_and(jnp.logical_not(sup_i), kcount < _NMS_TOPK)
            sigi = sig[:, i:i + 1]
            czi, cyi, cxi = cz[:, i:i + 1], cy[:, i:i + 1], cx[:, i:i + 1]
            dzi, dyi, dxi = dz[:, i:i + 1], dy[:, i:i + 1], dx[:, i:i + 1]
            lzi, hzi = czi - dzi * 0.5, czi + dzi * 0.5
            lyi, hyi = cyi - dyi * 0.5, cyi + dyi * 0.5
            lxi, hxi = cxi - dxi * 0.5, cxi + dxi * 0.5
            iz = jnp.maximum(jnp.minimum(hzi, hiz) - jnp.maximum(lzi, loz), 0.0)
            iy = jnp.maximum(jnp.minimum(hyi, hiy) - jnp.maximum(lyi, loy), 0.0)
            ix = jnp.maximum(jnp.minimum(hxi, hix) - jnp.maximum(lxi, lox), 0.0)
            inter = iz * iy * ix
            voli = dzi * dyi * dxi
            iou = inter / (voli + vols - inter + 1e-8)
            supnew = jnp.logical_or(jnp.logical_or(sup, iou > _NMS_THRESHOLD),
                                    lane60 == i)
            sup = jnp.logical_or(jnp.logical_and(take, supnew),
                                 jnp.logical_and(jnp.logical_not(take), sup))
            wm = jnp.logical_and(take, lane60 == kcount)
            vals_i = (1.0, sigi, czi, cyi, cxi, dzi, dyi, dxi)
            outs = [jnp.where(wm, v, o) for v, o in zip(vals_i, outs)]
            kcount = kcount + take.astype(jnp.int32)

        for c in range(8):
            out_ref[c] = outs[c]

    return body, N


def kernel(Cls, Shape, Offset):
    bs = Cls.shape[0]
    d, h, w = Cls.shape[2], Cls.shape[3], Cls.shape[4]
    body, N = _make_body(bs, d, h, w)

    zz, yy, xx = np.meshgrid(np.arange(d, dtype=np.float32),
                             np.arange(h, dtype=np.float32),
                             np.arange(w, dtype=np.float32), indexing='ij')
    anch_np = np.stack([zz.ravel(), yy.ravel(), xx.ravel()], axis=0)

    cls2 = Cls.reshape(bs, N)
    off2 = Offset.reshape(bs, 3, N)
    shp2 = Shape.reshape(bs, 3, N)
    out = pl.pallas_call(
        body,
        out_shape=jax.ShapeDtypeStruct((8, bs, _TOPK), jnp.float32),
    )(cls2, off2, shp2, jnp.asarray(anch_np))
    return jnp.transpose(out, (1, 2, 0))
